# baseline (speedup 1.0000x reference)
import functools

import jax
import jax.numpy as jnp
from jax.experimental import pallas as pl
from jax.experimental.pallas import tpu as pltpu

MEMN = 200000
RD = 256
BLK = 4000
NBLK = MEMN // BLK
NEG = -1.0e30


def _dotT(a, b):
    return jax.lax.dot_general(
        a, b, (((1,), (1,)), ((), ())),
        precision=jax.lax.Precision.HIGHEST,
        preferred_element_type=jnp.float32)


def _body(state_r, sp_w1_r, sp_b1_r, sp_w2_r, sp_b2_r, cont_w_r, cont_b_r,
          dir_w_r, dir_b_r, step_w_r, step_b_r, val_w_r, val_b_r,
          tp_w1_r, tp_b1_r, tp_w2_r, tp_b2_r, bank_r, vals_r, g_r, bank_hbm,
          obank_r, ovals_r, olat_r, onp_r, oact_r, olp_r, oval_r, oent_r,
          ws_r, rows_r, rs_r, ns_r, dma_sem):
    step = pl.program_id(0)
    b = jax.lax.rem(step + 1, NBLK)

    @pl.when(step == 0)
    def _init():
        h = jnp.maximum(_dotT(state_r[...], sp_w1_r[...]) + sp_b1_r[...], 0.0)
        rs = _dotT(h, sp_w2_r[...]) + sp_b2_r[...]
        rs_r[...] = rs
        nrm = jnp.sqrt(jnp.sum(rs * rs))
        ns_r[...] = rs / jnp.maximum(nrm, 1e-12)

    blk = bank_r[...]
    obank_r[...] = blk
    vals = vals_r[0]
    ovals_r[0] = vals

    ns = ns_r[...]
    sims = _dotT(ns, blk)
    ones = jnp.ones((1, RD), jnp.float32)
    sq = _dotT(ones, blk * blk)
    rn = jnp.sqrt(sq)
    w = sims / jnp.maximum(rn, 1e-12) * (vals + 1e-8)
    ws_r[pl.ds(b, 1), :] = w

    @pl.when(step == NBLK - 1)
    def _final():
        ws = ws_r[...]
        ri = jax.lax.broadcasted_iota(jnp.int32, (NBLK, BLK), 0)
        ci = jax.lax.broadcasted_iota(jnp.int32, (NBLK, BLK), 1)
        gidx = ri * BLK + ci
        for t in range(3):
            m = jnp.max(ws)
            idx = jnp.min(jnp.where(ws == m, gidx, MEMN))
            ws = jnp.where(gidx == idx, NEG, ws)
            cp = pltpu.make_async_copy(
                bank_hbm.at[pl.ds(idx, 1), :],
                rows_r.at[pl.ds(t, 1), :], dma_sem)
            cp.start()
            cp.wait()
        retrieved = (rows_r[0:1, :] + rows_r[1:2, :] + rows_r[2:3, :]) * (1.0 / 3.0)
        rs_f = 0.5 * rs_r[...] + 0.5 * retrieved

        logits = _dotT(rs_f, cont_w_r[...]) + cont_b_r[...]
        l8 = jax.lax.broadcasted_iota(jnp.int32, (1, 8), 1)
        valid = l8 < 2
        lm = jnp.where(valid, logits, NEG)
        mx = jnp.max(lm)
        ex = jnp.where(valid, jnp.exp(lm - mx), 0.0)
        se = jnp.sum(ex)
        probs = ex / se
        logz = jnp.log(se) + mx
        logp = logits - logz
        ent = -jnp.sum(jnp.where(valid, probs * logp, 0.0))
        ga = jnp.where(valid, logits + g_r[...], NEG)
        gmx = jnp.max(ga)
        act = jnp.min(jnp.where(ga == gmx, l8, 8))
        lp = jnp.sum(jnp.where(l8 == act, logp, 0.0))

        dirv = _dotT(rs_f, dir_w_r[...]) + dir_b_r[...]
        dn = jnp.sqrt(jnp.sum(dirv * dirv))
        dirn = dirv / jnp.maximum(dn, 1e-12)
        s_pre = jnp.sum(rs_f * step_w_r[...]) + step_b_r[0, 0]
        step_v = jax.nn.sigmoid(s_pre) * 2.0
        val_s = jnp.sum(rs_f * val_w_r[...]) + val_b_r[0, 0]
        nxt = rs_f + step_v * dirn

        h2 = jnp.maximum(_dotT(nxt, tp_w1_r[...]) + tp_b1_r[...], 0.0)
        lat = _dotT(h2, tp_w2_r[...]) + tp_b2_r[...]

        olat_r[...] = lat
        onp_r[...] = nxt
        oact_r[...] = jnp.full((1, 1), act, jnp.int32)
        olp_r[...] = jnp.full((1, 1), lp, jnp.float32)
        oval_r[...] = jnp.full((1, 1), val_s, jnp.float32)
        oent_r[...] = jnp.full((1, 1), ent, jnp.float32)
        obank_r[0:1, :] = nxt
        lv = jax.lax.broadcasted_iota(jnp.int32, (1, BLK), 1)
        ovals_r[0] = jnp.where(lv == 0, val_s, vals)


def kernel(state, sp_w1, sp_b1, sp_w2, sp_b2, cont_w, cont_b, dir_w, dir_b,
           step_w, step_b, val_w, val_b, tp_w1, tp_b1, tp_w2, tp_b2,
           memory_bank, memory_values):
    f32 = jnp.float32
    cont_w8 = jnp.zeros((8, RD), f32).at[:2].set(cont_w)
    cont_b8 = jnp.zeros((1, 8), f32).at[0, :2].set(cont_b)
    g = jax.random.gumbel(jax.random.key(42), (1, 2), dtype=f32)
    g8 = jnp.zeros((1, 8), f32).at[0, :2].set(g[0])
    vals3 = memory_values.reshape(NBLK, 1, BLK)

    def cm(shape):
        return pl.BlockSpec(shape, lambda i: (0,) * len(shape))

    bank_spec = pl.BlockSpec((BLK, RD), lambda i: ((i + 1) % NBLK, 0))
    vals_spec = pl.BlockSpec((1, 1, BLK), lambda i: ((i + 1) % NBLK, 0, 0))

    out_shape = (
        jax.ShapeDtypeStruct((MEMN, RD), f32),
        jax.ShapeDtypeStruct((NBLK, 1, BLK), f32),
        jax.ShapeDtypeStruct((1, 4096), f32),
        jax.ShapeDtypeStruct((1, RD), f32),
        jax.ShapeDtypeStruct((1, 1), jnp.int32),
        jax.ShapeDtypeStruct((1, 1), f32),
        jax.ShapeDtypeStruct((1, 1), f32),
        jax.ShapeDtypeStruct((1, 1), f32),
    )
    out_specs = (
        bank_spec,
        vals_spec,
        cm((1, 4096)),
        cm((1, RD)),
        cm((1, 1)),
        cm((1, 1)),
        cm((1, 1)),
        cm((1, 1)),
    )
    in_specs = [
        cm((1, 4096)),
        cm((1024, 4096)),
        cm((1, 1024)),
        cm((RD, 1024)),
        cm((1, RD)),
        cm((8, RD)),
        cm((1, 8)),
        cm((RD, RD)),
        cm((1, RD)),
        cm((1, RD)),
        cm((1, 1)),
        cm((1, RD)),
        cm((1, 1)),
        cm((1024, RD)),
        cm((1, 1024)),
        cm((4096, 1024)),
        cm((1, 4096)),
        bank_spec,
        vals_spec,
        cm((1, 8)),
        pl.BlockSpec(memory_space=pl.ANY),
    ]

    outs = pl.pallas_call(
        _body,
        grid=(NBLK,),
        in_specs=in_specs,
        out_specs=out_specs,
        out_shape=out_shape,
        scratch_shapes=[
            pltpu.VMEM((NBLK, BLK), f32),
            pltpu.VMEM((8, RD), f32),
            pltpu.VMEM((1, RD), f32),
            pltpu.VMEM((1, RD), f32),
            pltpu.SemaphoreType.DMA,
        ],
        compiler_params=pltpu.CompilerParams(
            dimension_semantics=("arbitrary",)),
    )(state, sp_w1, sp_b1.reshape(1, 1024), sp_w2, sp_b2.reshape(1, RD),
      cont_w8, cont_b8, dir_w, dir_b.reshape(1, RD), step_w,
      step_b.reshape(1, 1), val_w, val_b.reshape(1, 1), tp_w1,
      tp_b1.reshape(1, 1024), tp_w2, tp_b2.reshape(1, 4096),
      memory_bank, vals3, g8, memory_bank)

    (new_bank, new_vals3, latent, next_pos, act, lp, val, ent) = outs
    return (latent, next_pos, act.reshape(1).astype(jnp.int32),
            lp.reshape(1), val.reshape(1), ent.reshape(1),
            new_bank, new_vals3.reshape(MEMN))


# speedup vs baseline: 1.0000x; 1.0000x over previous
import functools

import jax
import jax.numpy as jnp
from jax.experimental import pallas as pl
from jax.experimental.pallas import tpu as pltpu

MEMN = 200000
RD = 256
BLK = 4000
NBLK = MEMN // BLK
NEG = -1.0e30


def _dotT(a, b):
    return jax.lax.dot_general(
        a, b, (((1,), (1,)), ((), ())),
        precision=jax.lax.Precision.HIGHEST,
        preferred_element_type=jnp.float32)


def _body(state_r, sp_w1_r, sp_b1_r, sp_w2_r, sp_b2_r, cont_w_r, cont_b_r,
          dir_w_r, dir_b_r, step_w_r, step_b_r, val_w_r, val_b_r,
          tp_w1_r, tp_b1_r, tp_w2_r, tp_b2_r, bank_r, vals_r, g_r, bank_hbm,
          obank_r, ovals_r, olat_r, onp_r, oact_r, olp_r, oval_r, oent_r,
          cols_r, ids_r, rows_r, rs_r, ns_r, dma_sem):
    step = pl.program_id(0)
    b = jax.lax.rem(step + 1, NBLK)

    @pl.when(step == 0)
    def _init():
        h = jnp.maximum(_dotT(state_r[...], sp_w1_r[...]) + sp_b1_r[...], 0.0)
        rs = _dotT(h, sp_w2_r[...]) + sp_b2_r[...]
        rs_r[...] = rs
        nrm = jnp.sqrt(jnp.sum(rs * rs))
        ns_r[...] = rs / jnp.maximum(nrm, 1e-12)
        cols_r[...] = jnp.full_like(cols_r, NEG)
        ids_r[...] = jnp.zeros_like(ids_r)

    blk = bank_r[...]
    obank_r[...] = blk
    vals = vals_r[0]
    ovals_r[0] = vals

    ns = ns_r[...]
    sims = _dotT(ns, blk)
    ones = jnp.ones((1, RD), jnp.float32)
    sq = _dotT(ones, blk * blk)
    rn = jnp.sqrt(sq)
    w = sims / jnp.maximum(rn, 1e-12) * (vals + 1e-8)

    c0, c1, c2 = cols_r[0:1, :], cols_r[1:2, :], cols_r[2:3, :]
    i0, i1, i2 = ids_r[0:1, :], ids_r[1:2, :], ids_r[2:3, :]
    bv = jnp.full((1, BLK), 0, jnp.int32) + b
    m0, m1, m2 = w > c0, w > c1, w > c2
    cols_r[0:1, :] = jnp.where(m0, w, c0)
    ids_r[0:1, :] = jnp.where(m0, bv, i0)
    cols_r[1:2, :] = jnp.where(m0, c0, jnp.where(m1, w, c1))
    ids_r[1:2, :] = jnp.where(m0, i0, jnp.where(m1, bv, i1))
    cols_r[2:3, :] = jnp.where(m1, c1, jnp.where(m2, w, c2))
    ids_r[2:3, :] = jnp.where(m1, i1, jnp.where(m2, bv, i2))

    @pl.when(step == NBLK - 1)
    def _final():
        ci = jax.lax.broadcasted_iota(jnp.int32, (1, BLK), 1)
        planes = [(cols_r[k:k + 1, :], ids_r[k:k + 1, :] * BLK + ci)
                  for k in range(3)]
        for t in range(3):
            m = planes[0][0]
            for cv, _ in planes[1:]:
                m = jnp.maximum(m, cv)
            ms = jnp.max(m)
            idx = MEMN
            for cv, gi in planes:
                idx = jnp.minimum(idx, jnp.min(jnp.where(cv == ms, gi, MEMN)))
            planes = [(jnp.where(gi == idx, NEG, cv), gi) for cv, gi in planes]
            cp = pltpu.make_async_copy(
                bank_hbm.at[pl.ds(idx, 1), :],
                rows_r.at[pl.ds(t, 1), :], dma_sem)
            cp.start()
            cp.wait()
        retrieved = (rows_r[0:1, :] + rows_r[1:2, :] + rows_r[2:3, :]) * (1.0 / 3.0)
        rs_f = 0.5 * rs_r[...] + 0.5 * retrieved

        logits = _dotT(rs_f, cont_w_r[...]) + cont_b_r[...]
        l8 = jax.lax.broadcasted_iota(jnp.int32, (1, 8), 1)
        valid = l8 < 2
        lm = jnp.where(valid, logits, NEG)
        mx = jnp.max(lm)
        ex = jnp.where(valid, jnp.exp(lm - mx), 0.0)
        se = jnp.sum(ex)
        probs = ex / se
        logz = jnp.log(se) + mx
        logp = logits - logz
        ent = -jnp.sum(jnp.where(valid, probs * logp, 0.0))
        ga = jnp.where(valid, logits + g_r[...], NEG)
        gmx = jnp.max(ga)
        act = jnp.min(jnp.where(ga == gmx, l8, 8))
        lp = jnp.sum(jnp.where(l8 == act, logp, 0.0))

        dirv = _dotT(rs_f, dir_w_r[...]) + dir_b_r[...]
        dn = jnp.sqrt(jnp.sum(dirv * dirv))
        dirn = dirv / jnp.maximum(dn, 1e-12)
        s_pre = jnp.sum(rs_f * step_w_r[...]) + step_b_r[0, 0]
        step_v = jax.nn.sigmoid(s_pre) * 2.0
        val_s = jnp.sum(rs_f * val_w_r[...]) + val_b_r[0, 0]
        nxt = rs_f + step_v * dirn

        h2 = jnp.maximum(_dotT(nxt, tp_w1_r[...]) + tp_b1_r[...], 0.0)
        lat = _dotT(h2, tp_w2_r[...]) + tp_b2_r[...]

        olat_r[...] = lat
        onp_r[...] = nxt
        oact_r[...] = jnp.full((1, 1), act, jnp.int32)
        olp_r[...] = jnp.full((1, 1), lp, jnp.float32)
        oval_r[...] = jnp.full((1, 1), val_s, jnp.float32)
        oent_r[...] = jnp.full((1, 1), ent, jnp.float32)
        obank_r[0:1, :] = nxt
        lv = jax.lax.broadcasted_iota(jnp.int32, (1, BLK), 1)
        ovals_r[0] = jnp.where(lv == 0, val_s, vals)


def kernel(state, sp_w1, sp_b1, sp_w2, sp_b2, cont_w, cont_b, dir_w, dir_b,
           step_w, step_b, val_w, val_b, tp_w1, tp_b1, tp_w2, tp_b2,
           memory_bank, memory_values):
    f32 = jnp.float32
    cont_w8 = jnp.zeros((8, RD), f32).at[:2].set(cont_w)
    cont_b8 = jnp.zeros((1, 8), f32).at[0, :2].set(cont_b)
    g = jax.random.gumbel(jax.random.key(42), (1, 2), dtype=f32)
    g8 = jnp.zeros((1, 8), f32).at[0, :2].set(g[0])
    vals3 = memory_values.reshape(NBLK, 1, BLK)

    def cm(shape):
        return pl.BlockSpec(shape, lambda i: (0,) * len(shape))

    bank_spec = pl.BlockSpec((BLK, RD), lambda i: ((i + 1) % NBLK, 0))
    vals_spec = pl.BlockSpec((1, 1, BLK), lambda i: ((i + 1) % NBLK, 0, 0))

    out_shape = (
        jax.ShapeDtypeStruct((MEMN, RD), f32),
        jax.ShapeDtypeStruct((NBLK, 1, BLK), f32),
        jax.ShapeDtypeStruct((1, 4096), f32),
        jax.ShapeDtypeStruct((1, RD), f32),
        jax.ShapeDtypeStruct((1, 1), jnp.int32),
        jax.ShapeDtypeStruct((1, 1), f32),
        jax.ShapeDtypeStruct((1, 1), f32),
        jax.ShapeDtypeStruct((1, 1), f32),
    )
    out_specs = (
        bank_spec,
        vals_spec,
        cm((1, 4096)),
        cm((1, RD)),
        cm((1, 1)),
        cm((1, 1)),
        cm((1, 1)),
        cm((1, 1)),
    )
    in_specs = [
        cm((1, 4096)),
        cm((1024, 4096)),
        cm((1, 1024)),
        cm((RD, 1024)),
        cm((1, RD)),
        cm((8, RD)),
        cm((1, 8)),
        cm((RD, RD)),
        cm((1, RD)),
        cm((1, RD)),
        cm((1, 1)),
        cm((1, RD)),
        cm((1, 1)),
        cm((1024, RD)),
        cm((1, 1024)),
        cm((4096, 1024)),
        cm((1, 4096)),
        bank_spec,
        vals_spec,
        cm((1, 8)),
        pl.BlockSpec(memory_space=pl.ANY),
    ]

    outs = pl.pallas_call(
        _body,
        grid=(NBLK,),
        in_specs=in_specs,
        out_specs=out_specs,
        out_shape=out_shape,
        scratch_shapes=[
            pltpu.VMEM((8, BLK), f32),
            pltpu.VMEM((8, BLK), jnp.int32),
            pltpu.VMEM((8, RD), f32),
            pltpu.VMEM((1, RD), f32),
            pltpu.VMEM((1, RD), f32),
            pltpu.SemaphoreType.DMA,
        ],
        compiler_params=pltpu.CompilerParams(
            dimension_semantics=("arbitrary",)),
    )(state, sp_w1, sp_b1.reshape(1, 1024), sp_w2, sp_b2.reshape(1, RD),
      cont_w8, cont_b8, dir_w, dir_b.reshape(1, RD), step_w,
      step_b.reshape(1, 1), val_w, val_b.reshape(1, 1), tp_w1,
      tp_b1.reshape(1, 1024), tp_w2, tp_b2.reshape(1, 4096),
      memory_bank, vals3, g8, memory_bank)

    (new_bank, new_vals3, latent, next_pos, act, lp, val, ent) = outs
    return (latent, next_pos, act.reshape(1).astype(jnp.int32),
            lp.reshape(1), val.reshape(1), ent.reshape(1),
            new_bank, new_vals3.reshape(MEMN))


# speedup vs baseline: 2.3538x; 2.3537x over previous
import functools

import jax
import jax.numpy as jnp
from jax.experimental import pallas as pl
from jax.experimental.pallas import tpu as pltpu

MEMN = 200000
RD = 256
BLK = 4000
NBLK = MEMN // BLK
NEG = -1.0e30


def _dotT(a, b):
    return jax.lax.dot_general(
        a, b, (((1,), (1,)), ((), ())),
        preferred_element_type=jnp.float32)


def _body(state_r, sp_w1_r, sp_b1_r, sp_w2_r, sp_b2_r, cont_w_r, cont_b_r,
          dir_w_r, dir_b_r, step_w_r, step_b_r, val_w_r, val_b_r,
          tp_w1_r, tp_b1_r, tp_w2_r, tp_b2_r, bank_r, vals_r, g_r, bank_hbm,
          obank_r, ovals_r, olat_r, onp_r, oact_r, olp_r, oval_r, oent_r,
          cols_r, ids_r, rows_r, rs_r, ns_r, dma_sem):
    step = pl.program_id(0)
    b = jax.lax.rem(step + 1, NBLK)

    @pl.when(step == 0)
    def _init():
        h = jnp.maximum(_dotT(state_r[...], sp_w1_r[...]) + sp_b1_r[...], 0.0)
        rs = _dotT(h, sp_w2_r[...]) + sp_b2_r[...]
        rs_r[...] = rs
        nrm = jnp.sqrt(jnp.sum(rs * rs))
        ns_r[...] = rs / jnp.maximum(nrm, 1e-12)
        cols_r[...] = jnp.full_like(cols_r, NEG)
        ids_r[...] = jnp.zeros_like(ids_r)

    blk = bank_r[...]
    obank_r[...] = blk
    vals = vals_r[0]
    ovals_r[0] = vals

    ns = ns_r[...]
    sims = _dotT(ns, blk)
    ones = jnp.ones((1, RD), jnp.float32)
    sq = _dotT(ones, blk * blk)
    rn = jnp.sqrt(sq)
    w = sims / jnp.maximum(rn, 1e-12) * (vals + 1e-8)

    c0, c1, c2 = cols_r[0:1, :], cols_r[1:2, :], cols_r[2:3, :]
    i0, i1, i2 = ids_r[0:1, :], ids_r[1:2, :], ids_r[2:3, :]
    bv = jnp.full((1, BLK), 0, jnp.int32) + b
    m0, m1, m2 = w > c0, w > c1, w > c2
    cols_r[0:1, :] = jnp.where(m0, w, c0)
    ids_r[0:1, :] = jnp.where(m0, bv, i0)
    cols_r[1:2, :] = jnp.where(m0, c0, jnp.where(m1, w, c1))
    ids_r[1:2, :] = jnp.where(m0, i0, jnp.where(m1, bv, i1))
    cols_r[2:3, :] = jnp.where(m1, c1, jnp.where(m2, w, c2))
    ids_r[2:3, :] = jnp.where(m1, i1, jnp.where(m2, bv, i2))

    @pl.when(step == NBLK - 1)
    def _final():
        ci = jax.lax.broadcasted_iota(jnp.int32, (1, BLK), 1)
        planes = [(cols_r[k:k + 1, :], ids_r[k:k + 1, :] * BLK + ci)
                  for k in range(3)]
        for t in range(3):
            m = planes[0][0]
            for cv, _ in planes[1:]:
                m = jnp.maximum(m, cv)
            ms = jnp.max(m)
            idx = MEMN
            for cv, gi in planes:
                idx = jnp.minimum(idx, jnp.min(jnp.where(cv == ms, gi, MEMN)))
            planes = [(jnp.where(gi == idx, NEG, cv), gi) for cv, gi in planes]
            cp = pltpu.make_async_copy(
                bank_hbm.at[pl.ds(idx, 1), :],
                rows_r.at[pl.ds(t, 1), :], dma_sem)
            cp.start()
            cp.wait()
        retrieved = (rows_r[0:1, :] + rows_r[1:2, :] + rows_r[2:3, :]) * (1.0 / 3.0)
        rs_f = 0.5 * rs_r[...] + 0.5 * retrieved

        logits = _dotT(rs_f, cont_w_r[...]) + cont_b_r[...]
        l8 = jax.lax.broadcasted_iota(jnp.int32, (1, 8), 1)
        valid = l8 < 2
        lm = jnp.where(valid, logits, NEG)
        mx = jnp.max(lm)
        ex = jnp.where(valid, jnp.exp(lm - mx), 0.0)
        se = jnp.sum(ex)
        probs = ex / se
        logz = jnp.log(se) + mx
        logp = logits - logz
        ent = -jnp.sum(jnp.where(valid, probs * logp, 0.0))
        ga = jnp.where(valid, logits + g_r[...], NEG)
        gmx = jnp.max(ga)
        act = jnp.min(jnp.where(ga == gmx, l8, 8))
        lp = jnp.sum(jnp.where(l8 == act, logp, 0.0))

        dirv = _dotT(rs_f, dir_w_r[...]) + dir_b_r[...]
        dn = jnp.sqrt(jnp.sum(dirv * dirv))
        dirn = dirv / jnp.maximum(dn, 1e-12)
        s_pre = jnp.sum(rs_f * step_w_r[...]) + step_b_r[0, 0]
        step_v = jax.nn.sigmoid(s_pre) * 2.0
        val_s = jnp.sum(rs_f * val_w_r[...]) + val_b_r[0, 0]
        nxt = rs_f + step_v * dirn

        h2 = jnp.maximum(_dotT(nxt, tp_w1_r[...]) + tp_b1_r[...], 0.0)
        lat = _dotT(h2, tp_w2_r[...]) + tp_b2_r[...]

        olat_r[...] = lat
        onp_r[...] = nxt
        oact_r[...] = jnp.full((1, 1), act, jnp.int32)
        olp_r[...] = jnp.full((1, 1), lp, jnp.float32)
        oval_r[...] = jnp.full((1, 1), val_s, jnp.float32)
        oent_r[...] = jnp.full((1, 1), ent, jnp.float32)
        obank_r[0:1, :] = nxt
        lv = jax.lax.broadcasted_iota(jnp.int32, (1, BLK), 1)
        ovals_r[0] = jnp.where(lv == 0, val_s, vals)


def kernel(state, sp_w1, sp_b1, sp_w2, sp_b2, cont_w, cont_b, dir_w, dir_b,
           step_w, step_b, val_w, val_b, tp_w1, tp_b1, tp_w2, tp_b2,
           memory_bank, memory_values):
    f32 = jnp.float32
    cont_w8 = jnp.zeros((8, RD), f32).at[:2].set(cont_w)
    cont_b8 = jnp.zeros((1, 8), f32).at[0, :2].set(cont_b)
    g = jax.random.gumbel(jax.random.key(42), (1, 2), dtype=f32)
    g8 = jnp.zeros((1, 8), f32).at[0, :2].set(g[0])
    vals3 = memory_values.reshape(NBLK, 1, BLK)

    def cm(shape):
        return pl.BlockSpec(shape, lambda i: (0,) * len(shape))

    bank_spec = pl.BlockSpec((BLK, RD), lambda i: ((i + 1) % NBLK, 0))
    vals_spec = pl.BlockSpec((1, 1, BLK), lambda i: ((i + 1) % NBLK, 0, 0))

    out_shape = (
        jax.ShapeDtypeStruct((MEMN, RD), f32),
        jax.ShapeDtypeStruct((NBLK, 1, BLK), f32),
        jax.ShapeDtypeStruct((1, 4096), f32),
        jax.ShapeDtypeStruct((1, RD), f32),
        jax.ShapeDtypeStruct((1, 1), jnp.int32),
        jax.ShapeDtypeStruct((1, 1), f32),
        jax.ShapeDtypeStruct((1, 1), f32),
        jax.ShapeDtypeStruct((1, 1), f32),
    )
    out_specs = (
        bank_spec,
        vals_spec,
        cm((1, 4096)),
        cm((1, RD)),
        cm((1, 1)),
        cm((1, 1)),
        cm((1, 1)),
        cm((1, 1)),
    )
    in_specs = [
        cm((1, 4096)),
        cm((1024, 4096)),
        cm((1, 1024)),
        cm((RD, 1024)),
        cm((1, RD)),
        cm((8, RD)),
        cm((1, 8)),
        cm((RD, RD)),
        cm((1, RD)),
        cm((1, RD)),
        cm((1, 1)),
        cm((1, RD)),
        cm((1, 1)),
        cm((1024, RD)),
        cm((1, 1024)),
        cm((4096, 1024)),
        cm((1, 4096)),
        bank_spec,
        vals_spec,
        cm((1, 8)),
        pl.BlockSpec(memory_space=pl.ANY),
    ]

    outs = pl.pallas_call(
        _body,
        grid=(NBLK,),
        in_specs=in_specs,
        out_specs=out_specs,
        out_shape=out_shape,
        scratch_shapes=[
            pltpu.VMEM((8, BLK), f32),
            pltpu.VMEM((8, BLK), jnp.int32),
            pltpu.VMEM((8, RD), f32),
            pltpu.VMEM((1, RD), f32),
            pltpu.VMEM((1, RD), f32),
            pltpu.SemaphoreType.DMA,
        ],
        compiler_params=pltpu.CompilerParams(
            dimension_semantics=("arbitrary",)),
    )(state, sp_w1, sp_b1.reshape(1, 1024), sp_w2, sp_b2.reshape(1, RD),
      cont_w8, cont_b8, dir_w, dir_b.reshape(1, RD), step_w,
      step_b.reshape(1, 1), val_w, val_b.reshape(1, 1), tp_w1,
      tp_b1.reshape(1, 1024), tp_w2, tp_b2.reshape(1, 4096),
      memory_bank, vals3, g8, memory_bank)

    (new_bank, new_vals3, latent, next_pos, act, lp, val, ent) = outs
    return (latent, next_pos, act.reshape(1).astype(jnp.int32),
            lp.reshape(1), val.reshape(1), ent.reshape(1),
            new_bank, new_vals3.reshape(MEMN))


# speedup vs baseline: 2.3589x; 1.0022x over previous
import jax
import jax.numpy as jnp
from jax.experimental import pallas as pl
from jax.experimental.pallas import tpu as pltpu

MEMN = 200000
RD = 256
BLK = 10000
NBLK = MEMN // BLK
NEG = -1.0e30


def _dotT(a, b):
    return jax.lax.dot_general(
        a, b, (((1,), (1,)), ((), ())),
        preferred_element_type=jnp.float32)


def _proj_body(state_r, w1_r, b1_r, w2_r, b2_r, rs_r, ns_r):
    h = jnp.maximum(_dotT(state_r[...], w1_r[...]) + b1_r[...], 0.0)
    rs = _dotT(h, w2_r[...]) + b2_r[...]
    rs_r[...] = rs
    nrm = jnp.sqrt(jnp.sum(rs * rs))
    ns_r[...] = rs / jnp.maximum(nrm, 1e-12)


def _scan_body(ns_r, bank_r, vals_r, bank_hbm,
               obank_r, ovals_r, ret_r,
               cols_r, ids_r, rows_r, dma_sem):
    step = pl.program_id(0)
    b = step

    @pl.when(step == 0)
    def _init():
        cols_r[...] = jnp.full_like(cols_r, NEG)
        ids_r[...] = jnp.zeros_like(ids_r)

    blk = bank_r[...]
    obank_r[...] = blk
    vals = vals_r[0]
    ovals_r[0] = vals

    ns = ns_r[...]
    sims = _dotT(ns, blk)
    ones = jnp.ones((1, RD), jnp.float32)
    sq = _dotT(ones, blk * blk)
    rn = jnp.sqrt(sq)
    w = sims / jnp.maximum(rn, 1e-12) * (vals + 1e-8)

    c0, c1, c2 = cols_r[0:1, :], cols_r[1:2, :], cols_r[2:3, :]
    i0, i1, i2 = ids_r[0:1, :], ids_r[1:2, :], ids_r[2:3, :]
    bv = jnp.full((1, BLK), 0, jnp.int32) + b
    m0, m1, m2 = w > c0, w > c1, w > c2
    cols_r[0:1, :] = jnp.where(m0, w, c0)
    ids_r[0:1, :] = jnp.where(m0, bv, i0)
    cols_r[1:2, :] = jnp.where(m0, c0, jnp.where(m1, w, c1))
    ids_r[1:2, :] = jnp.where(m0, i0, jnp.where(m1, bv, i1))
    cols_r[2:3, :] = jnp.where(m1, c1, jnp.where(m2, w, c2))
    ids_r[2:3, :] = jnp.where(m1, i1, jnp.where(m2, bv, i2))

    @pl.when(step == NBLK - 1)
    def _final():
        ci = jax.lax.broadcasted_iota(jnp.int32, (1, BLK), 1)
        planes = [(cols_r[k:k + 1, :], ids_r[k:k + 1, :] * BLK + ci)
                  for k in range(3)]
        for t in range(3):
            m = planes[0][0]
            for cv, _ in planes[1:]:
                m = jnp.maximum(m, cv)
            ms = jnp.max(m)
            idx = MEMN
            for cv, gi in planes:
                idx = jnp.minimum(idx, jnp.min(jnp.where(cv == ms, gi, MEMN)))
            planes = [(jnp.where(gi == idx, NEG, cv), gi) for cv, gi in planes]
            cp = pltpu.make_async_copy(
                bank_hbm.at[pl.ds(idx, 1), :],
                rows_r.at[pl.ds(t, 1), :], dma_sem)
            cp.start()
            cp.wait()
        ret_r[...] = (rows_r[0:1, :] + rows_r[1:2, :] + rows_r[2:3, :]) * (1.0 / 3.0)


def _heads_body(rs_r, ret_r, cont_w_r, cont_b_r, dir_w_r, dir_b_r,
                step_w_r, step_b_r, val_w_r, val_b_r,
                tp_w1_r, tp_b1_r, tp_w2_r, tp_b2_r, g_r, bank_r, vals_r,
                olat_r, onp_r, oact_r, olp_r, oval_r, oent_r,
                obank_r, ovals_r):
    rs_f = 0.5 * rs_r[...] + 0.5 * ret_r[...]

    logits = _dotT(rs_f, cont_w_r[...]) + cont_b_r[...]
    l8 = jax.lax.broadcasted_iota(jnp.int32, (1, 8), 1)
    valid = l8 < 2
    lm = jnp.where(valid, logits, NEG)
    mx = jnp.max(lm)
    ex = jnp.where(valid, jnp.exp(lm - mx), 0.0)
    se = jnp.sum(ex)
    probs = ex / se
    logz = jnp.log(se) + mx
    logp = logits - logz
    ent = -jnp.sum(jnp.where(valid, probs * logp, 0.0))
    ga = jnp.where(valid, logits + g_r[...], NEG)
    gmx = jnp.max(ga)
    act = jnp.min(jnp.where(ga == gmx, l8, 8))
    lp = jnp.sum(jnp.where(l8 == act, logp, 0.0))

    dirv = _dotT(rs_f, dir_w_r[...]) + dir_b_r[...]
    dn = jnp.sqrt(jnp.sum(dirv * dirv))
    dirn = dirv / jnp.maximum(dn, 1e-12)
    s_pre = jnp.sum(rs_f * step_w_r[...]) + step_b_r[0, 0]
    step_v = jax.nn.sigmoid(s_pre) * 2.0
    val_s = jnp.sum(rs_f * val_w_r[...]) + val_b_r[0, 0]
    nxt = rs_f + step_v * dirn

    h2 = jnp.maximum(_dotT(nxt, tp_w1_r[...]) + tp_b1_r[...], 0.0)
    lat = _dotT(h2, tp_w2_r[...]) + tp_b2_r[...]

    olat_r[...] = lat
    onp_r[...] = nxt
    oact_r[...] = jnp.full((1, 1), act, jnp.int32)
    olp_r[...] = jnp.full((1, 1), lp, jnp.float32)
    oval_r[...] = jnp.full((1, 1), val_s, jnp.float32)
    oent_r[...] = jnp.full((1, 1), ent, jnp.float32)
    blk0 = bank_r[...]
    obank_r[...] = blk0
    obank_r[0:1, :] = nxt
    lv = jax.lax.broadcasted_iota(jnp.int32, (1, BLK), 1)
    ovals_r[0] = jnp.where(lv == 0, val_s, vals_r[0])


def kernel(state, sp_w1, sp_b1, sp_w2, sp_b2, cont_w, cont_b, dir_w, dir_b,
           step_w, step_b, val_w, val_b, tp_w1, tp_b1, tp_w2, tp_b2,
           memory_bank, memory_values):
    f32 = jnp.float32
    cont_w8 = jnp.zeros((8, RD), f32).at[:2].set(cont_w)
    cont_b8 = jnp.zeros((1, 8), f32).at[0, :2].set(cont_b)
    g = jax.random.gumbel(jax.random.key(42), (1, 2), dtype=f32)
    g8 = jnp.zeros((1, 8), f32).at[0, :2].set(g[0])
    vals3 = memory_values.reshape(NBLK, 1, BLK)

    def cm(shape):
        return pl.BlockSpec(shape, lambda *_: (0,) * len(shape))

    rs, ns = pl.pallas_call(
        _proj_body,
        in_specs=[cm((1, 4096)), cm((1024, 4096)), cm((1, 1024)),
                  cm((RD, 1024)), cm((1, RD))],
        out_specs=(cm((1, RD)), cm((1, RD))),
        out_shape=(jax.ShapeDtypeStruct((1, RD), f32),
                   jax.ShapeDtypeStruct((1, RD), f32)),
    )(state, sp_w1, sp_b1.reshape(1, 1024), sp_w2, sp_b2.reshape(1, RD))

    bank_spec = pl.BlockSpec((BLK, RD), lambda i: (i, 0))
    vals_spec = pl.BlockSpec((1, 1, BLK), lambda i: (i, 0, 0))

    obank, ovals3, retrieved = pl.pallas_call(
        _scan_body,
        grid=(NBLK,),
        in_specs=[cm((1, RD)), bank_spec, vals_spec,
                  pl.BlockSpec(memory_space=pl.ANY)],
        out_specs=(bank_spec, vals_spec, cm((1, RD))),
        out_shape=(jax.ShapeDtypeStruct((MEMN, RD), f32),
                   jax.ShapeDtypeStruct((NBLK, 1, BLK), f32),
                   jax.ShapeDtypeStruct((1, RD), f32)),
        scratch_shapes=[
            pltpu.VMEM((8, BLK), f32),
            pltpu.VMEM((8, BLK), jnp.int32),
            pltpu.VMEM((8, RD), f32),
            pltpu.SemaphoreType.DMA,
        ],
        compiler_params=pltpu.CompilerParams(
            dimension_semantics=("arbitrary",)),
    )(ns, memory_bank, vals3, memory_bank)

    outs = pl.pallas_call(
        _heads_body,
        grid=(1,),
        in_specs=[cm((1, RD)), cm((1, RD)), cm((8, RD)), cm((1, 8)),
                  cm((RD, RD)), cm((1, RD)), cm((1, RD)), cm((1, 1)),
                  cm((1, RD)), cm((1, 1)), cm((1024, RD)), cm((1, 1024)),
                  cm((4096, 1024)), cm((1, 4096)), cm((1, 8)),
                  pl.BlockSpec((8, RD), lambda *_: (0, 0)),
                  pl.BlockSpec((1, 1, BLK), lambda *_: (0, 0, 0))],
        out_specs=(cm((1, 4096)), cm((1, RD)), cm((1, 1)), cm((1, 1)),
                   cm((1, 1)), cm((1, 1)),
                   pl.BlockSpec((8, RD), lambda *_: (0, 0)),
                   pl.BlockSpec((1, 1, BLK), lambda *_: (0, 0, 0))),
        out_shape=(jax.ShapeDtypeStruct((1, 4096), f32),
                   jax.ShapeDtypeStruct((1, RD), f32),
                   jax.ShapeDtypeStruct((1, 1), jnp.int32),
                   jax.ShapeDtypeStruct((1, 1), f32),
                   jax.ShapeDtypeStruct((1, 1), f32),
                   jax.ShapeDtypeStruct((1, 1), f32),
                   jax.ShapeDtypeStruct((MEMN, RD), f32),
                   jax.ShapeDtypeStruct((NBLK, 1, BLK), f32)),
        input_output_aliases={15: 6, 16: 7},
    )(rs, retrieved, cont_w8, cont_b8, dir_w, dir_b.reshape(1, RD),
      step_w, step_b.reshape(1, 1), val_w, val_b.reshape(1, 1),
      tp_w1, tp_b1.reshape(1, 1024), tp_w2, tp_b2.reshape(1, 4096),
      g8, obank, ovals3)

    (latent, next_pos, act, lp, val, ent, new_bank, new_vals3) = outs
    return (latent, next_pos, act.reshape(1).astype(jnp.int32),
            lp.reshape(1), val.reshape(1), ent.reshape(1),
            new_bank, new_vals3.reshape(MEMN))
